# baseline (speedup 1.0000x reference)
import jax
import jax.numpy as jnp
from jax.experimental import pallas as pl

N = 10000
K = 30
NP = 10112
BR = 128
NR = 10112
NM = 4
NQ = NP // NM
NH = NQ // 2
KPAD = 32


def _knn_body(pv_ref, qr_ref, pt_ref, qc_ref, p_ref, idx_ref, sum_ref):
    pv = pv_ref[...].astype(jnp.float32)
    qr = qr_ref[...]
    pt = pt_ref[...].astype(jnp.float32)
    qc = qc_ref[...]
    g = pv[:, 0:1] * pt[0:1, :]
    g = g + pv[:, 1:2] * pt[1:2, :]
    g = g + pv[:, 2:3] * pt[2:3, :]
    d2 = (qr + qc) - 2.0 * g

    npf = jnp.float32(NP)
    inf = jnp.float32(jnp.inf)
    base = jax.lax.broadcasted_iota(jnp.int32, (BR, NH), 1).astype(jnp.float32)
    lo, hi, cl, ch, cas, cbs = [], [], [], [], [], []
    for mch in range(NM):
        blk = d2[:, mch * NQ:(mch + 1) * NQ]
        a = blk[:, :NH]
        b = blk[:, NH:]
        ca = base + jnp.float32(mch * NQ)
        cb = ca + jnp.float32(NH)
        wb = b < a
        lo.append(jnp.where(wb, b, a))
        hi.append(jnp.where(wb, a, b))
        cl.append(jnp.where(wb, cb, ca))
        ch.append(jnp.where(wb, ca, cb))
        cas.append(ca)
        cbs.append(cb)

    cols = []
    for _ in range(K):
        mm, am = [], []
        for mch in range(NM):
            m_m = jnp.min(lo[mch], axis=1, keepdims=True)
            a_m = jnp.min(jnp.where(lo[mch] == m_m, cl[mch], npf),
                          axis=1, keepdims=True)
            mm.append(m_m)
            am.append(a_m)
        mmc = jnp.concatenate(mm, axis=1)
        amc = jnp.concatenate(am, axis=1)
        m = jnp.min(mmc, axis=1, keepdims=True)
        amin = jnp.min(jnp.where(mmc == m, amc, npf),
                       axis=1, keepdims=True)
        cols.append(amin)
        for mch in range(NM):
            eqs = cl[mch] == amin
            lo[mch] = jnp.where(eqs, hi[mch], lo[mch])
            cl[mch] = jnp.where(eqs, ch[mch], cl[mch])
            hi[mch] = jnp.where(eqs, inf, hi[mch])
    idxf = jnp.concatenate(cols, axis=1)
    idx = idxf.astype(jnp.int32)
    idx_ref[...] = jnp.concatenate(
        [idx, jnp.zeros((BR, KPAD - K), jnp.int32)], axis=1)
    one = jnp.float32(1.0)
    zero = jnp.float32(0.0)
    sel_parts = []
    for mch in range(NM):
        lof = lo[mch] != inf
        hif = hi[mch] != inf
        pres_a = ((cl[mch] == cas[mch]) & lof) | ((ch[mch] == cas[mch]) & hif)
        pres_b = ((cl[mch] == cbs[mch]) & lof) | ((ch[mch] == cbs[mch]) & hif)
        sel_parts.append(jnp.where(pres_a, zero, one))
        sel_parts.append(jnp.where(pres_b, zero, one))
    sel = jnp.concatenate(sel_parts, axis=1)
    sum_ref[...] = jnp.dot(sel, p_ref[...],
                           preferred_element_type=jnp.float32)


def kernel(c, u, s, embedding1, embedding2):
    points = jnp.stack([c, u, s], axis=1)
    n = points.shape[0]
    mean = jnp.mean(points, axis=0, keepdims=True)
    pc = points - mean
    cov = (pc.T @ pc) / (n - 1)
    vi = jnp.linalg.inv(cov)
    pv = points @ vi
    q = jnp.einsum('ij,ij->i', pv, points)

    padr = NR - n
    padc = NP - n
    pv_pad = jnp.concatenate(
        [pv.astype(jnp.bfloat16), jnp.zeros((padr, 3), jnp.bfloat16)], 0)
    qr = jnp.concatenate(
        [q, jnp.zeros((padr,), jnp.float32)], 0)[:, None]
    pt = jnp.concatenate(
        [points.T.astype(jnp.bfloat16), jnp.zeros((3, padc), jnp.bfloat16)], 1)
    qc = jnp.concatenate([q, jnp.full((padc,), 1e30, jnp.float32)], 0)[None, :]
    p_pad = jnp.concatenate([points, jnp.zeros((padc, 3), jnp.float32)], 0)

    idx_out, sum_out = pl.pallas_call(
        _knn_body,
        grid=(NR // BR,),
        in_specs=[
            pl.BlockSpec((BR, 3), lambda i: (i, 0)),
            pl.BlockSpec((BR, 1), lambda i: (i, 0)),
            pl.BlockSpec((3, NP), lambda i: (0, 0)),
            pl.BlockSpec((1, NP), lambda i: (0, 0)),
            pl.BlockSpec((NP, 3), lambda i: (0, 0)),
        ],
        out_specs=[
            pl.BlockSpec((BR, KPAD), lambda i: (i, 0)),
            pl.BlockSpec((BR, 3), lambda i: (i, 0)),
        ],
        out_shape=[
            jax.ShapeDtypeStruct((NR, KPAD), jnp.int32),
            jax.ShapeDtypeStruct((NR, 3), jnp.float32),
        ],
    )(pv_pad, qr, pt, qc, p_pad)

    indices = idx_out[:N, :K]
    out = sum_out[:N] / jnp.float32(K)
    return out, indices


# speedup vs baseline: 1.2849x; 1.2849x over previous
import jax
import jax.numpy as jnp
from jax.experimental import pallas as pl

N = 10000
K = 30
NP = 10112
BR = 256
NR = 10240
KPAD = 32


def _knn_body(pv_ref, qr_ref, pt_ref, qc_ref, p_ref, idx_ref, sum_ref):
    pv = pv_ref[...].astype(jnp.float32)
    qr = qr_ref[...]
    pt = pt_ref[...].astype(jnp.float32)
    qc = qc_ref[...]
    g = pv[:, 0:1] * pt[0:1, :]
    g = g + pv[:, 1:2] * pt[1:2, :]
    g = g + pv[:, 2:3] * pt[2:3, :]
    d2 = (qr + qc) - 2.0 * g

    colf = jax.lax.broadcasted_iota(jnp.int32, (BR, NP), 1).astype(jnp.float32)
    npf = jnp.float32(NP)
    inf = jnp.float32(jnp.inf)
    cols = []
    for _ in range(K):
        m = jnp.min(d2, axis=1, keepdims=True)
        amin = jnp.min(jnp.where(d2 == m, colf, npf),
                       axis=1, keepdims=True)
        cols.append(amin)
        d2 = jnp.where(colf == amin, inf, d2)
    idxf = jnp.concatenate(cols, axis=1)
    idx = idxf.astype(jnp.int32)
    idx_ref[...] = jnp.concatenate(
        [idx, jnp.zeros((BR, KPAD - K), jnp.int32)], axis=1)
    sel = (d2 == inf).astype(jnp.float32)
    sum_ref[...] = jnp.dot(sel, p_ref[...],
                           preferred_element_type=jnp.float32)


def kernel(c, u, s, embedding1, embedding2):
    points = jnp.stack([c, u, s], axis=1)
    n = points.shape[0]
    mean = jnp.mean(points, axis=0, keepdims=True)
    pc = points - mean
    cov = (pc.T @ pc) / (n - 1)
    vi = jnp.linalg.inv(cov)
    pv = points @ vi
    q = jnp.einsum('ij,ij->i', pv, points)

    padr = NR - n
    padc = NP - n
    pv_pad = jnp.concatenate(
        [pv.astype(jnp.bfloat16), jnp.zeros((padr, 3), jnp.bfloat16)], 0)
    qr = jnp.concatenate(
        [q, jnp.zeros((padr,), jnp.float32)], 0)[:, None]
    pt = jnp.concatenate(
        [points.T.astype(jnp.bfloat16), jnp.zeros((3, padc), jnp.bfloat16)], 1)
    qc = jnp.concatenate([q, jnp.full((padc,), 1e30, jnp.float32)], 0)[None, :]
    p_pad = jnp.concatenate([points, jnp.zeros((padc, 3), jnp.float32)], 0)

    idx_out, sum_out = pl.pallas_call(
        _knn_body,
        grid=(NR // BR,),
        in_specs=[
            pl.BlockSpec((BR, 3), lambda i: (i, 0)),
            pl.BlockSpec((BR, 1), lambda i: (i, 0)),
            pl.BlockSpec((3, NP), lambda i: (0, 0)),
            pl.BlockSpec((1, NP), lambda i: (0, 0)),
            pl.BlockSpec((NP, 3), lambda i: (0, 0)),
        ],
        out_specs=[
            pl.BlockSpec((BR, KPAD), lambda i: (i, 0)),
            pl.BlockSpec((BR, 3), lambda i: (i, 0)),
        ],
        out_shape=[
            jax.ShapeDtypeStruct((NR, KPAD), jnp.int32),
            jax.ShapeDtypeStruct((NR, 3), jnp.float32),
        ],
    )(pv_pad, qr, pt, qc, p_pad)

    indices = idx_out[:N, :K]
    out = sum_out[:N] / jnp.float32(K)
    return out, indices


# speedup vs baseline: 1.3558x; 1.0552x over previous
import jax
import jax.numpy as jnp
from jax.experimental import pallas as pl

N = 10000
K = 30
NP = 10112
BR = 512
NR = 10240
KPAD = 32


def _knn_body(pv_ref, qr_ref, pt_ref, qc_ref, p_ref, idx_ref, sum_ref):
    pv = pv_ref[...].astype(jnp.float32)
    qr = qr_ref[...]
    pt = pt_ref[...].astype(jnp.float32)
    qc = qc_ref[...]
    g = pv[:, 0:1] * pt[0:1, :]
    g = g + pv[:, 1:2] * pt[1:2, :]
    g = g + pv[:, 2:3] * pt[2:3, :]
    d2 = (qr + qc) - 2.0 * g

    colf = jax.lax.broadcasted_iota(jnp.int32, (BR, NP), 1).astype(jnp.float32)
    npf = jnp.float32(NP)
    inf = jnp.float32(jnp.inf)
    cols = []
    for _ in range(K):
        m = jnp.min(d2, axis=1, keepdims=True)
        amin = jnp.min(jnp.where(d2 == m, colf, npf),
                       axis=1, keepdims=True)
        cols.append(amin)
        d2 = jnp.where(colf == amin, inf, d2)
    idxf = jnp.concatenate(cols, axis=1)
    idx = idxf.astype(jnp.int32)
    idx_ref[...] = jnp.concatenate(
        [idx, jnp.zeros((BR, KPAD - K), jnp.int32)], axis=1)
    sel = (d2 == inf).astype(jnp.float32)
    sum_ref[...] = jnp.dot(sel, p_ref[...],
                           preferred_element_type=jnp.float32)


def kernel(c, u, s, embedding1, embedding2):
    points = jnp.stack([c, u, s], axis=1)
    n = points.shape[0]
    mean = jnp.mean(points, axis=0, keepdims=True)
    pc = points - mean
    cov = (pc.T @ pc) / (n - 1)
    vi = jnp.linalg.inv(cov)
    pv = points @ vi
    q = jnp.einsum('ij,ij->i', pv, points)

    padr = NR - n
    padc = NP - n
    pv_pad = jnp.concatenate(
        [pv.astype(jnp.bfloat16), jnp.zeros((padr, 3), jnp.bfloat16)], 0)
    qr = jnp.concatenate(
        [q, jnp.zeros((padr,), jnp.float32)], 0)[:, None]
    pt = jnp.concatenate(
        [points.T.astype(jnp.bfloat16), jnp.zeros((3, padc), jnp.bfloat16)], 1)
    qc = jnp.concatenate([q, jnp.full((padc,), 1e30, jnp.float32)], 0)[None, :]
    p_pad = jnp.concatenate([points, jnp.zeros((padc, 3), jnp.float32)], 0)

    idx_out, sum_out = pl.pallas_call(
        _knn_body,
        grid=(NR // BR,),
        in_specs=[
            pl.BlockSpec((BR, 3), lambda i: (i, 0)),
            pl.BlockSpec((BR, 1), lambda i: (i, 0)),
            pl.BlockSpec((3, NP), lambda i: (0, 0)),
            pl.BlockSpec((1, NP), lambda i: (0, 0)),
            pl.BlockSpec((NP, 3), lambda i: (0, 0)),
        ],
        out_specs=[
            pl.BlockSpec((BR, KPAD), lambda i: (i, 0)),
            pl.BlockSpec((BR, 3), lambda i: (i, 0)),
        ],
        out_shape=[
            jax.ShapeDtypeStruct((NR, KPAD), jnp.int32),
            jax.ShapeDtypeStruct((NR, 3), jnp.float32),
        ],
    )(pv_pad, qr, pt, qc, p_pad)

    indices = idx_out[:N, :K]
    out = sum_out[:N] / jnp.float32(K)
    return out, indices
